# baseline (speedup 1.0000x reference)
import dataclasses

import jax
import jax.numpy as jnp
from jax.experimental import pallas as pl
from jax.experimental.pallas import tpu as pltpu
from jax.experimental.pallas import tpu_sc as plsc

_B = 16
_N = 32768
_CH1 = _N // 16
_NV1 = _CH1 // 16
_CH2 = _N // 32
_NV2 = _CH2 // 16

_mesh = plsc.VectorSubcoreMesh(core_axis_name="c", subcore_axis_name="s")

_F = jnp.float32

_cp = pltpu.CompilerParams()
if "needs_layout_passes" in pltpu.CompilerParams.__dataclass_fields__:
    _cp = dataclasses.replace(_cp, needs_layout_passes=False)


def _vfull(v):
    return jnp.full((16,), v, dtype=_F)


@jax.named_call
def _sc_main(pred_flat, true_flat, z, seg):
    @pl.kernel(
        out_type=(jax.ShapeDtypeStruct((32, 4, 16, 16), _F),
                  jax.ShapeDtypeStruct((2, 32, 16), _F)),
        mesh=_mesh,
        compiler_params=_cp,
        scratch_types=[
            pltpu.VMEM((_CH1,), _F),
            pltpu.VMEM((_CH1,), jnp.int32),
            pltpu.VMEM((3 * _CH2,), _F),
            pltpu.VMEM((3 * _CH2,), _F),
            pltpu.VMEM((_CH2,), jnp.int32),
            pltpu.VMEM((32, 16), _F),
            pltpu.VMEM((4, 16), _F),
            pltpu.VMEM((4, 16, 16), _F),
            pltpu.SemaphoreType.DMA,
            pltpu.SemaphoreType.DMA,
            pltpu.SemaphoreType.DMA,
            pltpu.SemaphoreType.DMA,
            pltpu.SemaphoreType.DMA,
        ],
    )
    def body(pf_hbm, tf_hbm, z_hbm, seg_hbm, out_hbm, xch_hbm,
             z1_t, seg1_t, pp_t, tt_t, seg2_t, mm_t, thr_t, acc_t,
             sem1, sem2, sem3, sem4, sem5):
        core = jax.lax.axis_index("c")
        sub = jax.lax.axis_index("s")
        tile = core * 16 + sub
        a1 = sub * _CH1
        a2 = tile * _CH2

        cp1 = pltpu.async_copy(z_hbm.at[pl.ds(a1, _CH1)], z1_t, sem1)
        cp2 = pltpu.async_copy(seg_hbm.at[pl.ds(a1, _CH1)], seg1_t, sem2)
        cp3 = pltpu.async_copy(pf_hbm.at[pl.ds(3 * a2, 3 * _CH2)], pp_t, sem3)
        cp4 = pltpu.async_copy(tf_hbm.at[pl.ds(3 * a2, 3 * _CH2)], tt_t, sem4)
        cp5 = pltpu.async_copy(seg_hbm.at[pl.ds(a2, _CH2)], seg2_t, sem5)

        lane = jax.lax.iota(jnp.int32, 16)

        cp1.wait()
        cp2.wait()
        smin1 = seg1_t[pl.ds(0, 16)][0]
        smax1 = seg1_t[pl.ds(_CH1 - 16, 16)][15]

        def seg_mm(s, carry):
            rmin, rmax = carry

            def vec_mm(i, c):
                vmin, vmax = c
                sl = pl.ds(i * 16, 16)
                zv = z1_t[sl]
                m = seg1_t[sl] == s
                vmin = jnp.minimum(vmin, jnp.where(m, zv, _vfull(jnp.inf)))
                vmax = jnp.maximum(vmax, jnp.where(m, zv, _vfull(-jnp.inf)))
                return vmin, vmax

            vmin, vmax = jax.lax.fori_loop(
                0, _NV1, vec_mm, (_vfull(jnp.inf), _vfull(-jnp.inf)))
            here = lane == s
            rmin = jnp.where(here, jnp.broadcast_to(jnp.min(vmin), (16,)), rmin)
            rmax = jnp.where(here, jnp.broadcast_to(jnp.max(vmax), (16,)), rmax)
            return rmin, rmax

        rmin, rmax = jax.lax.fori_loop(
            smin1, smax1 + 1, seg_mm, (_vfull(jnp.inf), _vfull(-jnp.inf)))

        thr_t[0] = rmin
        thr_t[1] = rmax
        cpa = pltpu.async_copy(thr_t.at[0], xch_hbm.at[core, sub], sem1)
        cpb = pltpu.async_copy(thr_t.at[1], xch_hbm.at[core, 16 + sub], sem2)
        cpa.wait()
        cpb.wait()
        plsc.subcore_barrier()
        pltpu.sync_copy(xch_hbm.at[core], mm_t)

        def red_mm(k, carry):
            mn, mx = carry
            return (jnp.minimum(mn, mm_t[k]), jnp.maximum(mx, mm_t[16 + k]))

        mn, mx = jax.lax.fori_loop(0, 16, red_mm,
                                   (_vfull(jnp.inf), _vfull(-jnp.inf)))
        rng = mx - mn
        zb = mn + 0.405 * rng
        zt = mx - 0.405 * rng
        mid = (mn + mx) / 2.0
        half = 0.19 * (zt - zb) / 2.0
        thr_t[0] = zb
        thr_t[1] = zt
        thr_t[2] = mid - half
        thr_t[3] = mid + half

        for q in range(4):
            for s in range(16):
                acc_t[q, s] = _vfull(0.0)

        cp3.wait()
        cp4.wait()
        cp5.wait()
        smin2 = seg2_t[pl.ds(0, 16)][0]
        smax2 = seg2_t[pl.ds(_CH2 - 16, 16)][15]
        gi0 = lane * 3
        gi1 = gi0 + 1
        gi2 = gi0 + 2

        def vec_sum(j, _):
            base = j * 48
            i0 = gi0 + base
            i1 = gi1 + base
            i2 = gi2 + base
            pxv = plsc.load_gather(pp_t, [i0])
            pyv = plsc.load_gather(pp_t, [i1])
            pzv = plsc.load_gather(pp_t, [i2])
            txv = plsc.load_gather(tt_t, [i0])
            tyv = plsc.load_gather(tt_t, [i1])
            tzv = plsc.load_gather(tt_t, [i2])
            sv = seg2_t[pl.ds(j * 16, 16)]
            dx = pxv - txv
            dy = pyv - tyv
            dz = pzv - tzv

            def hub1(d):
                ad = jnp.abs(d)
                return jnp.where(ad < 0.5, 0.5 * d * d, 0.5 * (ad - 0.25))

            hub = hub1(dx) + hub1(dy) + hub1(dz)
            sq = dx * dx + dy * dy + dz * dz
            zv = tzv
            zbv = plsc.load_gather(thr_t.at[0], [sv])
            ztv = plsc.load_gather(thr_t.at[1], [sv])
            fbv = plsc.load_gather(thr_t.at[2], [sv])
            ftv = plsc.load_gather(thr_t.at[3], [sv])
            fil = (zv >= zbv) & (zv <= ztv) & (zv >= fbv) & (zv <= ftv)
            filf = jnp.where(fil, _vfull(1.0), _vfull(0.0))
            hf = filf * hub
            se = (1.0 - filf) * sq
            z16 = _vfull(0.0)

            def seg_acc(s, _):
                m = sv == s
                acc_t[0, s] = acc_t[0, s] + jnp.where(m, hf, z16)
                acc_t[1, s] = acc_t[1, s] + jnp.where(m, filf, z16)
                acc_t[2, s] = acc_t[2, s] + jnp.where(m, se, z16)
                acc_t[3, s] = acc_t[3, s] + jnp.where(m, _vfull(1.0), z16)
                return 0

            jax.lax.fori_loop(smin2, smax2 + 1, seg_acc, 0)
            return 0

        jax.lax.fori_loop(0, _NV2, vec_sum, 0)
        pltpu.sync_copy(acc_t, out_hbm.at[tile])

    return body(pred_flat, true_flat, z, seg)


@jax.named_call
def _sc_final(parts):
    @pl.kernel(
        out_type=jax.ShapeDtypeStruct((16,), _F),
        mesh=_mesh,
        compiler_params=_cp,
        scratch_types=[
            pltpu.VMEM((32768,), _F),
            pltpu.VMEM((16,), _F),
        ],
    )
    def body(part_hbm, out_hbm, part_t, out_t):
        core = jax.lax.axis_index("c")
        sub = jax.lax.axis_index("s")
        tile = core * 16 + sub
        lane = jax.lax.iota(jnp.int32, 16)

        def do_final(_):
            pltpu.sync_copy(part_hbm, part_t)
            z16 = _vfull(0.0)
            totals = []
            for q in range(4):
                res = z16
                for s in range(16):
                    def red(k, v):
                        return v + part_t[pl.ds((k * 64 + q * 16 + s) * 16, 16)]
                    v = jax.lax.fori_loop(0, 32, red, z16)
                    res = jnp.where(lane == s,
                                    jnp.broadcast_to(jnp.sum(v), (16,)), res)
                totals.append(res)
            fs, fc, es, cm = totals
            ec = cm - fc
            fil_mean = jnp.where(fc > 0, fs / (3.0 * jnp.maximum(fc, 1.0)), z16)
            ele_mean = jnp.where(ec > 0, es / (3.0 * jnp.maximum(ec, 1.0)), z16)
            loss = (50.0 / _B) * jnp.sum(fil_mean) + (1.0 / _B) * jnp.sum(ele_mean)
            out_t[...] = jnp.broadcast_to(loss, (16,))
            pltpu.sync_copy(out_t, out_hbm)
            return 0

        jax.lax.cond(tile == 0, do_final, lambda _: 0, 0)

    return body(parts.reshape(32768))


def _fin_body(parts_ref, out_ref):
    x = parts_ref[...]
    fs = jnp.sum(x[:, 0], axis=(0, 2))
    fc = jnp.sum(x[:, 1], axis=(0, 2))
    es = jnp.sum(x[:, 2], axis=(0, 2))
    cm = jnp.sum(x[:, 3], axis=(0, 2))
    ec = cm - fc
    zero = jnp.zeros((16,), _F)
    fil_mean = jnp.where(fc > 0, fs / (3.0 * jnp.maximum(fc, 1.0)), zero)
    ele_mean = jnp.where(ec > 0, es / (3.0 * jnp.maximum(ec, 1.0)), zero)
    loss = (50.0 / _B) * jnp.sum(fil_mean) + (1.0 / _B) * jnp.sum(ele_mean)
    out_ref[...] = jnp.reshape(loss, (1, 1))


@jax.jit
def kernel(pred_coords, true_coords, batch_vector):
    pred_flat = pred_coords.reshape(3 * _N)
    true_flat = true_coords.reshape(3 * _N)
    z = true_coords[:, 2]
    seg = batch_vector.astype(jnp.int32)
    parts, _ = _sc_main(pred_flat, true_flat, z, seg)
    out = _sc_final(parts)
    return out[0]


# speedup vs baseline: 1.9232x; 1.9232x over previous
import dataclasses

import jax
import jax.numpy as jnp
from jax.experimental import pallas as pl
from jax.experimental.pallas import tpu as pltpu
from jax.experimental.pallas import tpu_sc as plsc

_B = 16
_N = 32768
_CH1 = _N // 16
_NV1 = _CH1 // 16
_CH2 = _N // 32
_NV2 = _CH2 // 16

_mesh = plsc.VectorSubcoreMesh(core_axis_name="c", subcore_axis_name="s")

_F = jnp.float32

_cp = pltpu.CompilerParams()
if "needs_layout_passes" in pltpu.CompilerParams.__dataclass_fields__:
    _cp = dataclasses.replace(_cp, needs_layout_passes=False)


def _vfull(v):
    return jnp.full((16,), v, dtype=_F)


@jax.named_call
def _sc_main(pred_pl, true_pl, seg):
    @pl.kernel(
        out_type=(jax.ShapeDtypeStruct((32, 4, 16, 16), _F),
                  jax.ShapeDtypeStruct((2, 32, 16), _F)),
        mesh=_mesh,
        compiler_params=_cp,
        scratch_types=[
            pltpu.VMEM((_CH1,), _F),
            pltpu.VMEM((_CH1,), jnp.int32),
            pltpu.VMEM((_CH2,), _F),
            pltpu.VMEM((_CH2,), _F),
            pltpu.VMEM((_CH2,), _F),
            pltpu.VMEM((_CH2,), _F),
            pltpu.VMEM((_CH2,), _F),
            pltpu.VMEM((_CH2,), _F),
            pltpu.VMEM((_CH2,), jnp.int32),
            pltpu.VMEM((32, 16), _F),
            pltpu.VMEM((4, 16), _F),
            pltpu.VMEM((4, 16, 16), _F),
            pltpu.SemaphoreType.DMA,
            pltpu.SemaphoreType.DMA,
            pltpu.SemaphoreType.DMA,
            pltpu.SemaphoreType.DMA,
            pltpu.SemaphoreType.DMA,
            pltpu.SemaphoreType.DMA,
            pltpu.SemaphoreType.DMA,
            pltpu.SemaphoreType.DMA,
            pltpu.SemaphoreType.DMA,
        ],
    )
    def body(pf_hbm, tf_hbm, seg_hbm, out_hbm, xch_hbm,
             z1_t, seg1_t, px_t, py_t, pz_t, tx_t, ty_t, tz_t, seg2_t,
             mm_t, thr_t, acc_t,
             sem1, sem2, sem3, sem4, sem5, sem6, sem7, sem8, sem9):
        core = jax.lax.axis_index("c")
        sub = jax.lax.axis_index("s")
        tile = core * 16 + sub
        a1 = sub * _CH1
        a2 = tile * _CH2

        cp1 = pltpu.async_copy(tf_hbm.at[pl.ds(2 * _N + a1, _CH1)], z1_t, sem1)
        cp2 = pltpu.async_copy(seg_hbm.at[pl.ds(a1, _CH1)], seg1_t, sem2)
        cp3 = pltpu.async_copy(pf_hbm.at[pl.ds(a2, _CH2)], px_t, sem3)
        cp4 = pltpu.async_copy(pf_hbm.at[pl.ds(_N + a2, _CH2)], py_t, sem4)
        cp5 = pltpu.async_copy(pf_hbm.at[pl.ds(2 * _N + a2, _CH2)], pz_t, sem5)
        cp6 = pltpu.async_copy(tf_hbm.at[pl.ds(a2, _CH2)], tx_t, sem6)
        cp7 = pltpu.async_copy(tf_hbm.at[pl.ds(_N + a2, _CH2)], ty_t, sem7)
        cp8 = pltpu.async_copy(tf_hbm.at[pl.ds(2 * _N + a2, _CH2)], tz_t, sem8)
        cp9 = pltpu.async_copy(seg_hbm.at[pl.ds(a2, _CH2)], seg2_t, sem9)

        lane = jax.lax.iota(jnp.int32, 16)

        cp1.wait()
        cp2.wait()
        smin1 = seg1_t[pl.ds(0, 16)][0]
        smax1 = seg1_t[pl.ds(_CH1 - 16, 16)][15]

        def seg_mm(s, carry):
            rmin, rmax = carry

            def vec_mm(i, c):
                vmin, vmax = c
                sl = pl.ds(i * 16, 16)
                zv = z1_t[sl]
                m = seg1_t[sl] == s
                vmin = jnp.minimum(vmin, jnp.where(m, zv, _vfull(jnp.inf)))
                vmax = jnp.maximum(vmax, jnp.where(m, zv, _vfull(-jnp.inf)))
                return vmin, vmax

            vmin, vmax = jax.lax.fori_loop(
                0, _NV1, vec_mm, (_vfull(jnp.inf), _vfull(-jnp.inf)))
            here = lane == s
            rmin = jnp.where(here, jnp.broadcast_to(jnp.min(vmin), (16,)), rmin)
            rmax = jnp.where(here, jnp.broadcast_to(jnp.max(vmax), (16,)), rmax)
            return rmin, rmax

        rmin, rmax = jax.lax.fori_loop(
            smin1, smax1 + 1, seg_mm, (_vfull(jnp.inf), _vfull(-jnp.inf)))

        thr_t[0] = rmin
        thr_t[1] = rmax
        cpa = pltpu.async_copy(thr_t.at[0], xch_hbm.at[core, sub], sem1)
        cpb = pltpu.async_copy(thr_t.at[1], xch_hbm.at[core, 16 + sub], sem2)
        cpa.wait()
        cpb.wait()
        plsc.subcore_barrier()
        pltpu.sync_copy(xch_hbm.at[core], mm_t)

        def red_mm(k, carry):
            mn, mx = carry
            return (jnp.minimum(mn, mm_t[k]), jnp.maximum(mx, mm_t[16 + k]))

        mn, mx = jax.lax.fori_loop(0, 16, red_mm,
                                   (_vfull(jnp.inf), _vfull(-jnp.inf)))
        rng = mx - mn
        zb = mn + 0.405 * rng
        zt = mx - 0.405 * rng
        mid = (mn + mx) / 2.0
        half = 0.19 * (zt - zb) / 2.0
        thr_t[0] = zb
        thr_t[1] = zt
        thr_t[2] = mid - half
        thr_t[3] = mid + half

        for q in range(4):
            for s in range(16):
                acc_t[q, s] = _vfull(0.0)

        cp3.wait()
        cp4.wait()
        cp5.wait()
        cp6.wait()
        cp7.wait()
        cp8.wait()
        cp9.wait()
        smin2 = seg2_t[pl.ds(0, 16)][0]
        smax2 = seg2_t[pl.ds(_CH2 - 16, 16)][15]

        def vec_sum(j, _):
            sl = pl.ds(j * 16, 16)
            pxv = px_t[sl]
            pyv = py_t[sl]
            pzv = pz_t[sl]
            txv = tx_t[sl]
            tyv = ty_t[sl]
            tzv = tz_t[sl]
            sv = seg2_t[sl]
            dx = pxv - txv
            dy = pyv - tyv
            dz = pzv - tzv

            def hub1(d):
                ad = jnp.abs(d)
                return jnp.where(ad < 0.5, 0.5 * d * d, 0.5 * (ad - 0.25))

            hub = hub1(dx) + hub1(dy) + hub1(dz)
            sq = dx * dx + dy * dy + dz * dz
            zv = tzv
            zbv = plsc.load_gather(thr_t.at[0], [sv])
            ztv = plsc.load_gather(thr_t.at[1], [sv])
            fbv = plsc.load_gather(thr_t.at[2], [sv])
            ftv = plsc.load_gather(thr_t.at[3], [sv])
            fil = (zv >= zbv) & (zv <= ztv) & (zv >= fbv) & (zv <= ftv)
            filf = jnp.where(fil, _vfull(1.0), _vfull(0.0))
            hf = filf * hub
            se = (1.0 - filf) * sq
            z16 = _vfull(0.0)

            def seg_acc(s, _):
                m = sv == s
                acc_t[0, s] = acc_t[0, s] + jnp.where(m, hf, z16)
                acc_t[1, s] = acc_t[1, s] + jnp.where(m, filf, z16)
                acc_t[2, s] = acc_t[2, s] + jnp.where(m, se, z16)
                acc_t[3, s] = acc_t[3, s] + jnp.where(m, _vfull(1.0), z16)
                return 0

            jax.lax.fori_loop(smin2, smax2 + 1, seg_acc, 0)
            return 0

        jax.lax.fori_loop(0, _NV2, vec_sum, 0)
        pltpu.sync_copy(acc_t, out_hbm.at[tile])

    return body(pred_pl, true_pl, seg)


@jax.named_call
def _sc_final(parts):
    @pl.kernel(
        out_type=jax.ShapeDtypeStruct((16,), _F),
        mesh=_mesh,
        compiler_params=_cp,
        scratch_types=[
            pltpu.VMEM((32768,), _F),
            pltpu.VMEM((16,), _F),
        ],
    )
    def body(part_hbm, out_hbm, part_t, out_t):
        core = jax.lax.axis_index("c")
        sub = jax.lax.axis_index("s")
        tile = core * 16 + sub
        lane = jax.lax.iota(jnp.int32, 16)

        def do_final(_):
            pltpu.sync_copy(part_hbm, part_t)
            z16 = _vfull(0.0)
            totals = []
            for q in range(4):
                res = z16
                for s in range(16):
                    def red(k, v):
                        return v + part_t[pl.ds((k * 64 + q * 16 + s) * 16, 16)]
                    v = jax.lax.fori_loop(0, 32, red, z16)
                    res = jnp.where(lane == s,
                                    jnp.broadcast_to(jnp.sum(v), (16,)), res)
                totals.append(res)
            fs, fc, es, cm = totals
            ec = cm - fc
            fil_mean = jnp.where(fc > 0, fs / (3.0 * jnp.maximum(fc, 1.0)), z16)
            ele_mean = jnp.where(ec > 0, es / (3.0 * jnp.maximum(ec, 1.0)), z16)
            loss = (50.0 / _B) * jnp.sum(fil_mean) + (1.0 / _B) * jnp.sum(ele_mean)
            out_t[...] = jnp.broadcast_to(loss, (16,))
            pltpu.sync_copy(out_t, out_hbm)
            return 0

        jax.lax.cond(tile == 0, do_final, lambda _: 0, 0)

    return body(parts.reshape(32768))


def _fin_body(parts_ref, out_ref):
    x = parts_ref[...]
    fs = jnp.sum(x[:, 0], axis=(0, 2))
    fc = jnp.sum(x[:, 1], axis=(0, 2))
    es = jnp.sum(x[:, 2], axis=(0, 2))
    cm = jnp.sum(x[:, 3], axis=(0, 2))
    ec = cm - fc
    zero = jnp.zeros((16,), _F)
    fil_mean = jnp.where(fc > 0, fs / (3.0 * jnp.maximum(fc, 1.0)), zero)
    ele_mean = jnp.where(ec > 0, es / (3.0 * jnp.maximum(ec, 1.0)), zero)
    loss = (50.0 / _B) * jnp.sum(fil_mean) + (1.0 / _B) * jnp.sum(ele_mean)
    out_ref[...] = jnp.reshape(loss, (1, 1))


@jax.jit
def kernel(pred_coords, true_coords, batch_vector):
    pred_pl = pred_coords.T.reshape(3 * _N)
    true_pl = true_coords.T.reshape(3 * _N)
    seg = batch_vector.astype(jnp.int32)
    parts, _ = _sc_main(pred_pl, true_pl, seg)
    out = _sc_final(parts)
    return out[0]


# speedup vs baseline: 3.0910x; 1.6072x over previous
import dataclasses

import jax
import jax.numpy as jnp
from jax.experimental import pallas as pl
from jax.experimental.pallas import tpu as pltpu
from jax.experimental.pallas import tpu_sc as plsc

_B = 16
_N = 32768
_CH1 = _N // 16
_NV1 = _CH1 // 16
_CH2 = _N // 32
_NV2 = _CH2 // 16

_mesh = plsc.VectorSubcoreMesh(core_axis_name="c", subcore_axis_name="s")

_F = jnp.float32

_cp = pltpu.CompilerParams()
if "needs_layout_passes" in pltpu.CompilerParams.__dataclass_fields__:
    _cp = dataclasses.replace(_cp, needs_layout_passes=False)


def _vfull(v):
    return jnp.full((16,), v, dtype=_F)


@jax.named_call
def _sc_main(pred_pl, true_pl, seg):
    @pl.kernel(
        out_type=(jax.ShapeDtypeStruct((32, 4, 16, 16), _F),
                  jax.ShapeDtypeStruct((2, 32, 16), _F)),
        mesh=_mesh,
        compiler_params=_cp,
        scratch_types=[
            pltpu.VMEM((_CH1,), _F),
            pltpu.VMEM((_CH1,), jnp.int32),
            pltpu.VMEM((_CH2,), _F),
            pltpu.VMEM((_CH2,), _F),
            pltpu.VMEM((_CH2,), _F),
            pltpu.VMEM((_CH2,), _F),
            pltpu.VMEM((_CH2,), _F),
            pltpu.VMEM((_CH2,), _F),
            pltpu.VMEM((_CH2,), jnp.int32),
            pltpu.VMEM((32, 16), _F),
            pltpu.VMEM((4, 16), _F),
            pltpu.VMEM((4, 16, 16), _F),
            pltpu.SemaphoreType.DMA,
            pltpu.SemaphoreType.DMA,
            pltpu.SemaphoreType.DMA,
            pltpu.SemaphoreType.DMA,
            pltpu.SemaphoreType.DMA,
            pltpu.SemaphoreType.DMA,
            pltpu.SemaphoreType.DMA,
            pltpu.SemaphoreType.DMA,
            pltpu.SemaphoreType.DMA,
        ],
    )
    def body(pf_hbm, tf_hbm, seg_hbm, out_hbm, xch_hbm,
             z1_t, seg1_t, px_t, py_t, pz_t, tx_t, ty_t, tz_t, seg2_t,
             mm_t, thr_t, acc_t,
             sem1, sem2, sem3, sem4, sem5, sem6, sem7, sem8, sem9):
        core = jax.lax.axis_index("c")
        sub = jax.lax.axis_index("s")
        tile = core * 16 + sub
        a1 = sub * _CH1
        a2 = tile * _CH2

        cp1 = pltpu.async_copy(tf_hbm.at[pl.ds(2 * _N + a1, _CH1)], z1_t, sem1)
        cp2 = pltpu.async_copy(seg_hbm.at[pl.ds(a1, _CH1)], seg1_t, sem2)
        cp3 = pltpu.async_copy(pf_hbm.at[pl.ds(a2, _CH2)], px_t, sem3)
        cp4 = pltpu.async_copy(pf_hbm.at[pl.ds(_N + a2, _CH2)], py_t, sem4)
        cp5 = pltpu.async_copy(pf_hbm.at[pl.ds(2 * _N + a2, _CH2)], pz_t, sem5)
        cp6 = pltpu.async_copy(tf_hbm.at[pl.ds(a2, _CH2)], tx_t, sem6)
        cp7 = pltpu.async_copy(tf_hbm.at[pl.ds(_N + a2, _CH2)], ty_t, sem7)
        cp8 = pltpu.async_copy(tf_hbm.at[pl.ds(2 * _N + a2, _CH2)], tz_t, sem8)
        cp9 = pltpu.async_copy(seg_hbm.at[pl.ds(a2, _CH2)], seg2_t, sem9)

        lane = jax.lax.iota(jnp.int32, 16)

        cp1.wait()
        cp2.wait()
        smin1 = seg1_t[pl.ds(0, 16)][0]
        smax1 = seg1_t[pl.ds(_CH1 - 16, 16)][15]

        def seg_mm(s, carry):
            rmin, rmax = carry

            def vec_mm(i, c):
                vmin, vmax = c
                sl = pl.ds(i * 16, 16)
                zv = z1_t[sl]
                m = seg1_t[sl] == s
                vmin = jnp.minimum(vmin, jnp.where(m, zv, _vfull(jnp.inf)))
                vmax = jnp.maximum(vmax, jnp.where(m, zv, _vfull(-jnp.inf)))
                return vmin, vmax

            vmin, vmax = jax.lax.fori_loop(
                0, _NV1, vec_mm, (_vfull(jnp.inf), _vfull(-jnp.inf)))
            here = lane == s
            rmin = jnp.where(here, jnp.broadcast_to(jnp.min(vmin), (16,)), rmin)
            rmax = jnp.where(here, jnp.broadcast_to(jnp.max(vmax), (16,)), rmax)
            return rmin, rmax

        rmin, rmax = jax.lax.fori_loop(
            smin1, smax1 + 1, seg_mm, (_vfull(jnp.inf), _vfull(-jnp.inf)))

        thr_t[0] = rmin
        thr_t[1] = rmax
        cpa = pltpu.async_copy(thr_t.at[0], xch_hbm.at[core, sub], sem1)
        cpb = pltpu.async_copy(thr_t.at[1], xch_hbm.at[core, 16 + sub], sem2)
        cpa.wait()
        cpb.wait()
        plsc.subcore_barrier()
        pltpu.sync_copy(xch_hbm.at[core], mm_t)

        def red_mm(k, carry):
            mn, mx = carry
            return (jnp.minimum(mn, mm_t[k]), jnp.maximum(mx, mm_t[16 + k]))

        mn, mx = jax.lax.fori_loop(0, 16, red_mm,
                                   (_vfull(jnp.inf), _vfull(-jnp.inf)))
        rng = mx - mn
        zb = mn + 0.405 * rng
        zt = mx - 0.405 * rng
        mid = (mn + mx) / 2.0
        half = 0.19 * (zt - zb) / 2.0
        thr_t[0] = zb
        thr_t[1] = zt
        thr_t[2] = mid - half
        thr_t[3] = mid + half

        for q in range(4):
            for s in range(16):
                acc_t[q, s] = _vfull(0.0)

        cp3.wait()
        cp4.wait()
        cp5.wait()
        cp6.wait()
        cp7.wait()
        cp8.wait()
        cp9.wait()
        smin2 = seg2_t[pl.ds(0, 16)][0]
        smax2 = seg2_t[pl.ds(_CH2 - 16, 16)][15]

        def vec_sum(j, _):
            sl = pl.ds(j * 16, 16)
            pxv = px_t[sl]
            pyv = py_t[sl]
            pzv = pz_t[sl]
            txv = tx_t[sl]
            tyv = ty_t[sl]
            tzv = tz_t[sl]
            sv = seg2_t[sl]
            dx = pxv - txv
            dy = pyv - tyv
            dz = pzv - tzv

            def hub1(d):
                ad = jnp.abs(d)
                return jnp.where(ad < 0.5, 0.5 * d * d, 0.5 * (ad - 0.25))

            hub = hub1(dx) + hub1(dy) + hub1(dz)
            sq = dx * dx + dy * dy + dz * dz
            zv = tzv
            zbv = plsc.load_gather(thr_t.at[0], [sv])
            ztv = plsc.load_gather(thr_t.at[1], [sv])
            fbv = plsc.load_gather(thr_t.at[2], [sv])
            ftv = plsc.load_gather(thr_t.at[3], [sv])
            fil = (zv >= zbv) & (zv <= ztv) & (zv >= fbv) & (zv <= ftv)
            filf = jnp.where(fil, _vfull(1.0), _vfull(0.0))
            hf = filf * hub
            se = (1.0 - filf) * sq
            z16 = _vfull(0.0)

            def seg_acc(s, _):
                m = sv == s
                acc_t[0, s] = acc_t[0, s] + jnp.where(m, hf, z16)
                acc_t[1, s] = acc_t[1, s] + jnp.where(m, filf, z16)
                acc_t[2, s] = acc_t[2, s] + jnp.where(m, se, z16)
                acc_t[3, s] = acc_t[3, s] + jnp.where(m, _vfull(1.0), z16)
                return 0

            jax.lax.fori_loop(smin2, smax2 + 1, seg_acc, 0)
            return 0

        jax.lax.fori_loop(0, _NV2, vec_sum, 0)
        pltpu.sync_copy(acc_t, out_hbm.at[tile])

    return body(pred_pl, true_pl, seg)


@jax.named_call
def _sc_final(parts):
    @pl.kernel(
        out_type=jax.ShapeDtypeStruct((16,), _F),
        mesh=_mesh,
        compiler_params=_cp,
        scratch_types=[
            pltpu.VMEM((32768,), _F),
            pltpu.VMEM((16,), _F),
        ],
    )
    def body(part_hbm, out_hbm, part_t, out_t):
        core = jax.lax.axis_index("c")
        sub = jax.lax.axis_index("s")
        tile = core * 16 + sub
        lane = jax.lax.iota(jnp.int32, 16)

        def do_final(_):
            pltpu.sync_copy(part_hbm, part_t)
            z16 = _vfull(0.0)
            totals = []
            for q in range(4):
                res = z16
                for s in range(16):
                    def red(k, v):
                        return v + part_t[pl.ds((k * 64 + q * 16 + s) * 16, 16)]
                    v = jax.lax.fori_loop(0, 32, red, z16)
                    res = jnp.where(lane == s,
                                    jnp.broadcast_to(jnp.sum(v), (16,)), res)
                totals.append(res)
            fs, fc, es, cm = totals
            ec = cm - fc
            fil_mean = jnp.where(fc > 0, fs / (3.0 * jnp.maximum(fc, 1.0)), z16)
            ele_mean = jnp.where(ec > 0, es / (3.0 * jnp.maximum(ec, 1.0)), z16)
            loss = (50.0 / _B) * jnp.sum(fil_mean) + (1.0 / _B) * jnp.sum(ele_mean)
            out_t[...] = jnp.broadcast_to(loss, (16,))
            pltpu.sync_copy(out_t, out_hbm)
            return 0

        jax.lax.cond(tile == 0, do_final, lambda _: 0, 0)

    return body(parts.reshape(32768))


def _fin_body(parts_ref, out_ref):
    x = parts_ref[...]
    fs = jnp.sum(x[:, 0], axis=(0, 2))
    fc = jnp.sum(x[:, 1], axis=(0, 2))
    es = jnp.sum(x[:, 2], axis=(0, 2))
    cm = jnp.sum(x[:, 3], axis=(0, 2))
    ec = cm - fc
    zero = jnp.zeros((16,), _F)
    fil_mean = jnp.where(fc > 0, fs / (3.0 * jnp.maximum(fc, 1.0)), zero)
    ele_mean = jnp.where(ec > 0, es / (3.0 * jnp.maximum(ec, 1.0)), zero)
    loss = (50.0 / _B) * jnp.sum(fil_mean) + (1.0 / _B) * jnp.sum(ele_mean)
    out_ref[...] = jnp.reshape(loss, (1, 1))


@jax.jit
def kernel(pred_coords, true_coords, batch_vector):
    pred_pl = pred_coords.T.reshape(3 * _N)
    true_pl = true_coords.T.reshape(3 * _N)
    seg = batch_vector.astype(jnp.int32)
    parts, _ = _sc_main(pred_pl, true_pl, seg)
    out = pl.pallas_call(
        _fin_body,
        out_shape=jax.ShapeDtypeStruct((1, 1), _F),
    )(parts)
    return out[0, 0]
